# Initial kernel scaffold; baseline (speedup 1.0000x reference)
import functools

import jax
import jax.numpy as jnp
from jax import lax
from jax.experimental import pallas as pl
from jax.experimental.pallas import tpu as pltpu
from jax.experimental.pallas import tpu_sc as plsc

NUM_ACTIONS = 1000000
N_EMBED = 32
BATCH = 16384
HIST = 50

B = BATCH * HIST
NC, NS = 2, 16
NW = NC * NS
B_PER_W = B // NW
CHUNK = 3200
NCHUNK = B_PER_W // CHUNK

_mesh = plsc.VectorSubcoreMesh(core_axis_name="c", subcore_axis_name="s")


@functools.partial(
    pl.kernel,
    mesh=_mesh,
    out_type=jax.ShapeDtypeStruct((B, N_EMBED), jnp.float32),
    scratch_types=[
        pltpu.VMEM((CHUNK,), jnp.int32),
        pltpu.VMEM((CHUNK, N_EMBED), jnp.float32),
        pltpu.SemaphoreType.DMA,
    ],
)
def _gather_kernel(idx_hbm, table_hbm, out_hbm, idx_v, rows_v, sem):
    wid = lax.axis_index("s") * NC + lax.axis_index("c")
    base = wid * B_PER_W

    def body(i, carry):
        off = base + i * CHUNK
        pltpu.sync_copy(idx_hbm.at[pl.ds(off, CHUNK)], idx_v)
        pltpu.async_copy(table_hbm.at[idx_v], rows_v, sem).wait()
        pltpu.sync_copy(rows_v, out_hbm.at[pl.ds(off, CHUNK)])
        return carry

    lax.fori_loop(0, NCHUNK, body, 0)


def kernel(x, table):
    flat = x.reshape(-1).astype(jnp.int32)
    out = _gather_kernel(flat, table)
    return out.reshape(BATCH, HIST, N_EMBED)


# speedup vs baseline: 1.1117x; 1.1117x over previous
import functools

import jax
import jax.numpy as jnp
from jax import lax
from jax.experimental import pallas as pl
from jax.experimental.pallas import tpu as pltpu
from jax.experimental.pallas import tpu_sc as plsc

NUM_ACTIONS = 1000000
N_EMBED = 32
BATCH = 16384
HIST = 50

B = BATCH * HIST
NC, NS = 2, 16
NW = NC * NS
B_PER_W = B // NW
CHUNK = 3200
NCHUNK = B_PER_W // CHUNK

_mesh = plsc.VectorSubcoreMesh(core_axis_name="c", subcore_axis_name="s")


@functools.partial(
    pl.kernel,
    mesh=_mesh,
    out_type=jax.ShapeDtypeStruct((B, N_EMBED), jnp.float32),
    scratch_types=[
        pltpu.VMEM((CHUNK,), jnp.int32),
        pltpu.VMEM((CHUNK, N_EMBED), jnp.float32),
        pltpu.SemaphoreType.DMA,
    ],
    compiler_params=pltpu.CompilerParams(use_tc_tiling_on_sc=False),
)
def _gather_kernel(idx_hbm, table_hbm, out_hbm, idx_v, rows_v, sem):
    wid = lax.axis_index("s") * NC + lax.axis_index("c")
    base = wid * B_PER_W

    def body(i, carry):
        off = base + i * CHUNK
        pltpu.sync_copy(idx_hbm.at[pl.ds(off, CHUNK)], idx_v)
        pltpu.async_copy(table_hbm.at[idx_v], rows_v, sem).wait()
        pltpu.sync_copy(rows_v, out_hbm.at[pl.ds(off, CHUNK)])
        return carry

    lax.fori_loop(0, NCHUNK, body, 0)


def kernel(x, table):
    flat = x.reshape(-1).astype(jnp.int32)
    out = _gather_kernel(flat, table)
    return out.reshape(BATCH, HIST, N_EMBED)


# speedup vs baseline: 1.1127x; 1.0009x over previous
import functools

import jax
import jax.numpy as jnp
from jax import lax
from jax.experimental import pallas as pl
from jax.experimental.pallas import tpu as pltpu
from jax.experimental.pallas import tpu_sc as plsc

NUM_ACTIONS = 1000000
N_EMBED = 32
BATCH = 16384
HIST = 50

B = BATCH * HIST
NC, NS = 2, 16
NW = NC * NS
B_PER_W = B // NW
CHUNK = 1600
NCHUNK = B_PER_W // CHUNK
NPAIR = NCHUNK // 2

_mesh = plsc.VectorSubcoreMesh(core_axis_name="c", subcore_axis_name="s")


@functools.partial(
    pl.kernel,
    mesh=_mesh,
    out_type=jax.ShapeDtypeStruct((B, N_EMBED), jnp.float32),
    scratch_types=[
        pltpu.VMEM((CHUNK,), jnp.int32),
        pltpu.VMEM((CHUNK,), jnp.int32),
        pltpu.VMEM((CHUNK, N_EMBED), jnp.float32),
        pltpu.VMEM((CHUNK, N_EMBED), jnp.float32),
        pltpu.SemaphoreType.DMA,
        pltpu.SemaphoreType.DMA,
        pltpu.SemaphoreType.DMA,
        pltpu.SemaphoreType.DMA,
    ],
    compiler_params=pltpu.CompilerParams(use_tc_tiling_on_sc=False),
)
def _gather_kernel(idx_hbm, table_hbm, out_hbm, idx0, idx1, rows0, rows1,
                   gsem0, gsem1, ssem0, ssem1):
    wid = lax.axis_index("s") * NC + lax.axis_index("c")
    base = wid * B_PER_W
    bufs = ((idx0, rows0, gsem0, ssem0), (idx1, rows1, gsem1, ssem1))

    def pair(j, carry):
        for b, (idx_v, rows_v, gsem, ssem) in enumerate(bufs):
            off = base + (2 * j + b) * CHUNK

            @pl.when(j >= 1)
            def _wait_prev_store():
                pltpu.make_async_copy(
                    rows_v, out_hbm.at[pl.ds(0, CHUNK)], ssem).wait()

            pltpu.sync_copy(idx_hbm.at[pl.ds(off, CHUNK)], idx_v)
            pltpu.async_copy(table_hbm.at[idx_v], rows_v, gsem)

        for b, (idx_v, rows_v, gsem, ssem) in enumerate(bufs):
            off = base + (2 * j + b) * CHUNK
            pltpu.make_async_copy(table_hbm.at[idx_v], rows_v, gsem).wait()
            pltpu.async_copy(rows_v, out_hbm.at[pl.ds(off, CHUNK)], ssem)
        return carry

    lax.fori_loop(0, NPAIR, pair, 0)

    for idx_v, rows_v, gsem, ssem in bufs:
        pltpu.make_async_copy(rows_v, out_hbm.at[pl.ds(0, CHUNK)], ssem).wait()


def kernel(x, table):
    flat = x.reshape(-1).astype(jnp.int32)
    out = _gather_kernel(flat, table)
    return out.reshape(BATCH, HIST, N_EMBED)


# speedup vs baseline: 1.6166x; 1.4528x over previous
import functools

import jax
import jax.numpy as jnp
from jax import lax
from jax.experimental import pallas as pl
from jax.experimental.pallas import tpu as pltpu
from jax.experimental.pallas import tpu_sc as plsc

NUM_ACTIONS = 1000000
N_EMBED = 32
BATCH = 16384
HIST = 50

NC, NS = 2, 16
NW = NC * NS
N_PER_W = BATCH // NW
NB = 128
NBLK = N_PER_W // NB
NPAIR = HIST // 2

_mesh = plsc.VectorSubcoreMesh(core_axis_name="c", subcore_axis_name="s")


@functools.partial(
    pl.kernel,
    mesh=_mesh,
    out_type=jax.ShapeDtypeStruct((HIST, N_EMBED, BATCH), jnp.float32),
    scratch_types=[
        pltpu.VMEM((HIST, NB), jnp.int32),
        pltpu.VMEM((HIST, NB), jnp.int32),
        pltpu.VMEM((HIST, NB), jnp.int32),
        pltpu.VMEM((NB, 128), jnp.float32),
        pltpu.VMEM((NB, 128), jnp.float32),
        pltpu.VMEM((N_EMBED, NB), jnp.float32),
        pltpu.VMEM((N_EMBED, NB), jnp.float32),
        pltpu.SemaphoreType.DMA,
        pltpu.SemaphoreType.DMA,
        pltpu.SemaphoreType.DMA,
        pltpu.SemaphoreType.DMA,
    ],
    compiler_params=pltpu.CompilerParams(
        use_tc_tiling_on_sc=True, needs_layout_passes=False),
)
def _gather_kernel(xt_hbm, tab4_hbm, out_hbm, xt_v, idx4_v, rem32_v,
                   g0, g1, d0, d1, gsem0, gsem1, ssem0, ssem1):
    wid = lax.axis_index("s") * NC + lax.axis_index("c")
    nbase = wid * N_PER_W
    bufs = ((g0, d0, gsem0, ssem0), (g1, d1, gsem1, ssem1))
    lanes = lax.iota(jnp.int32, 16)

    def extract(g_v, d_v, h):
        for g in range(8):
            lvec = lanes + (16 * g)
            cols0 = rem32_v[h, pl.ds(16 * g, 16)]

            def e_body(e, carry):
                vals = plsc.load_gather(g_v, [lvec, cols0 + e])
                d_v[e, pl.ds(16 * g, 16)] = vals
                return carry

            lax.fori_loop(0, N_EMBED, e_body, 0)

    def nblk_body(nblk, carry):
        n0 = nbase + nblk * NB
        pltpu.sync_copy(xt_hbm.at[:, pl.ds(n0, NB)], xt_v)
        def prep_body(h, carry):
            for g in range(8):
                v = xt_v[h, pl.ds(16 * g, 16)]
                idx4_v[h, pl.ds(16 * g, 16)] = v >> 2
                rem32_v[h, pl.ds(16 * g, 16)] = (v & 3) << 5
            return carry

        lax.fori_loop(0, HIST, prep_body, 0)

        pltpu.async_copy(tab4_hbm.at[idx4_v.at[0]], g0, gsem0)
        pltpu.async_copy(tab4_hbm.at[idx4_v.at[1]], g1, gsem1)

        def pair_body(j, carry):
            for b, (g_v, d_v, gsem, ssem) in enumerate(bufs):
                h = 2 * j + b

                @pl.when(j >= 1)
                def _reclaim_d():
                    pltpu.make_async_copy(
                        d_v, out_hbm.at[0, :, pl.ds(n0, NB)], ssem).wait()

                pltpu.make_async_copy(
                    tab4_hbm.at[idx4_v.at[h]], g_v, gsem).wait()
                extract(g_v, d_v, h)
                pltpu.async_copy(
                    d_v, out_hbm.at[h, :, pl.ds(n0, NB)], ssem)

                @pl.when(h + 2 < HIST)
                def _next_gather():
                    pltpu.async_copy(
                        tab4_hbm.at[idx4_v.at[h + 2]], g_v, gsem)
            return carry

        lax.fori_loop(0, NPAIR, pair_body, 0)
        for _, d_v, _, ssem in bufs:
            pltpu.make_async_copy(
                d_v, out_hbm.at[0, :, pl.ds(n0, NB)], ssem).wait()
        return carry

    lax.fori_loop(0, NBLK, nblk_body, 0)


def kernel(x, table):
    xt = x.T.astype(jnp.int32)
    tab4 = table.reshape(NUM_ACTIONS // 4, 128)
    out_t = _gather_kernel(xt, tab4)
    return jnp.transpose(out_t, (2, 0, 1))
